# baseline (speedup 1.0000x reference)
import jax
import jax.numpy as jnp
from jax.experimental import pallas as pl


def _add_body(x_ref, t_ref, o_ref):
    o_ref[...] = x_ref[...] + t_ref[...][None, :, :]


def kernel(inputs, pos_table):
    B, S, D = inputs.shape
    BS = 256
    return pl.pallas_call(
        _add_body,
        grid=(S // BS,),
        in_specs=[
            pl.BlockSpec((B, BS, D), lambda i: (0, i, 0)),
            pl.BlockSpec((BS, D), lambda i: (i, 0)),
        ],
        out_specs=pl.BlockSpec((B, BS, D), lambda i: (0, i, 0)),
        out_shape=jax.ShapeDtypeStruct((B, S, D), inputs.dtype),
    )(inputs, pos_table)


# speedup vs baseline: 1.0294x; 1.0294x over previous
import jax
import jax.numpy as jnp
from jax.experimental import pallas as pl


def _add_body(x_ref, t_ref, o_ref):
    o_ref[...] = x_ref[...] + t_ref[...][None, :, :]


def kernel(inputs, pos_table):
    B, S, D = inputs.shape
    BS = 1024
    return pl.pallas_call(
        _add_body,
        grid=(S // BS,),
        in_specs=[
            pl.BlockSpec((B, BS, D), lambda i: (0, i, 0)),
            pl.BlockSpec((BS, D), lambda i: (i, 0)),
        ],
        out_specs=pl.BlockSpec((B, BS, D), lambda i: (0, i, 0)),
        out_shape=jax.ShapeDtypeStruct((B, S, D), inputs.dtype),
    )(inputs, pos_table)
